# baseline (speedup 1.0000x reference)
import functools

import jax
import jax.numpy as jnp
from jax import lax
from jax.experimental import pallas as pl
from jax.experimental.pallas import tpu as pltpu
from jax.experimental.pallas import tpu_sc as plsc

V = 10000
E = 320000
H = 128
NPROP = 5
VP = 10240
NW = 32
EPW = E // NW
K = 400
NCH = EPW // K
RPT = VP // 16
BM = 1024
EH = E // 2
BEH = 1000

f32 = jnp.float32
bf16 = jnp.bfloat16


def _sc_mesh():
    return plsc.VectorSubcoreMesh(core_axis_name="c", subcore_axis_name="s")


_SC_PARAMS = pltpu.CompilerParams(use_tc_tiling_on_sc=False)


KG = 200
NCHG = EPW // KG


def _gather_body(ap, bp, ii, io, m1,
                 ii_0, ii_1, io_0, io_1, a_0, a_1, b_0, b_1, o_0, o_1,
                 sii0, sii1, sio0, sio1, sa0, sa1, sb0, sb1, so0, so1):
    wid = lax.axis_index("s") * 2 + lax.axis_index("c")
    iiv = (ii_0, ii_1)
    iov = (io_0, io_1)
    av = (a_0, a_1)
    bv = (b_0, b_1)
    ov = (o_0, o_1)
    s_ii = (sii0, sii1)
    s_io = (sio0, sio1)
    s_a = (sa0, sa1)
    s_b = (sb0, sb1)
    s_o = (so0, so1)

    def base(c):
        return wid * EPW + c * KG

    dI = {}
    dG = {}
    dO = {}

    def issue_idx(c):
        j = c & 1
        dI[c] = (pltpu.async_copy(ii.at[pl.ds(base(c), KG)], iiv[j], s_ii[j]),
                 pltpu.async_copy(io.at[pl.ds(base(c), KG)], iov[j], s_io[j]))

    def issue_g(c):
        j = c & 1
        dG[c] = (pltpu.async_copy(ap.at[iiv[j]], av[j], s_a[j]),
                 pltpu.async_copy(bp.at[iov[j]], bv[j], s_b[j]))

    issue_idx(0)
    dI[0][0].wait()
    dI[0][1].wait()
    issue_g(0)
    if NCHG > 1:
        issue_idx(1)
    for c in range(NCHG):
        j = c & 1
        dG[c][0].wait()
        dG[c][1].wait()
        if c + 2 < NCHG:
            issue_idx(c + 2)
        if c + 1 < NCHG:
            dI[c + 1][0].wait()
            dI[c + 1][1].wait()
            issue_g(c + 1)
        jo = (c // 2) & 1
        roff = (c % 2) * (KG // 2)
        if c % 2 == 0 and c // 2 >= 2:
            dO[c // 2 - 2].wait()
        a_r, b_r, o_r = av[j], bv[j], ov[jo]

        def row(r, carry):
            for cc in range(8):
                sr = 2 * r + (1 if cc >= 4 else 0)
                sl = pl.ds(16 * (cc % 4), 16)
                o_r[roff + r, pl.ds(16 * cc, 16)] = a_r[sr, sl] + b_r[sr, sl]
            return carry

        lax.fori_loop(0, KG // 2, row, 0, unroll=2)
        if c % 2 == 1:
            dO[c // 2] = pltpu.async_copy(
                ov[jo], m1.at[pl.ds(base(c - 1) // 2, KG)], s_o[jo])
    dO[NCHG // 2 - 2].wait()
    dO[NCHG // 2 - 1].wait()


def _sc_gather(ap, bp, ii, io):
    gk = pl.kernel(
        _gather_body,
        out_type=jax.ShapeDtypeStruct((EH, 128), bf16),
        mesh=_sc_mesh(),
        compiler_params=_SC_PARAMS,
        scratch_types=(
            [pltpu.VMEM((KG,), jnp.int32)] * 4
            + [pltpu.VMEM((KG, 64), bf16)] * 4
            + [pltpu.VMEM((KG, 128), bf16)] * 2
            + [pltpu.SemaphoreType.DMA] * 10
        ),
    )
    return gk(ap, bp, ii, io)


def _zero_fill(buf, rows):
    def zrow(r, carry):
        for cc in range(buf.shape[1] // 16):
            buf[r, pl.ds(16 * cc, 16)] = jnp.zeros((16,), f32)
        return carry

    lax.fori_loop(0, rows, zrow, 0)


def _scatter_body(m2, ioe, ioo, s2o, m2_0, m2_1, ioe_0, ioe_1, ioo_0, ioo_1,
                  se_v, so_v, z_v, S_sh, sm0, sm1, se0, se1, so0, so1):
    sid = lax.axis_index("s")
    cid = lax.axis_index("c")
    wid = sid * 2 + cid
    m2v = (m2_0, m2_1)
    ioev = (ioe_0, ioe_1)
    ioov = (ioo_0, ioo_1)
    s_m = (sm0, sm1)
    s_e = (se0, se1)
    s_o = (so0, so1)
    _zero_fill(z_v, 64)
    for i in range(RPT // 64):
        pltpu.sync_copy(z_v, S_sh.at[pl.ds(sid * RPT + i * 64, 64)])
    plsc.subcore_barrier()

    def base(c):
        return wid * EPW + c * K

    dL = {}

    def load(c):
        j = c & 1
        hb = pl.multiple_of(base(c) // 2, 8)
        dL[c] = (pltpu.async_copy(m2.at[pl.ds(hb, K // 2)], m2v[j], s_m[j]),
                 pltpu.async_copy(ioe.at[pl.ds(hb, K // 2)], ioev[j], s_e[j]),
                 pltpu.async_copy(ioo.at[pl.ds(hb, K // 2)], ioov[j], s_o[j]))

    load(0)
    for c in range(NCH):
        j = c & 1
        for d in dL[c]:
            d.wait()
        if c + 1 < NCH:
            load(c + 1)
        m_r = m2v[j]

        def srow(r, carry):
            for cc in range(4):
                sl = pl.ds(16 * cc, 16)
                se_v[r, sl] = m_r[r, sl]
                so_v[r, sl] = m_r[r, pl.ds(64 + 16 * cc, 16)]
            return carry

        lax.fori_loop(0, K // 2, srow, 0, unroll=2)
        pltpu.sync_copy(se_v, S_sh.at[ioev[j]], add=True)
        pltpu.sync_copy(so_v, S_sh.at[ioov[j]], add=True)
    plsc.subcore_barrier()
    pltpu.sync_copy(S_sh.at[pl.ds(sid * RPT, RPT)], s2o.at[cid, pl.ds(sid * RPT, RPT)])


def _sc_scatter(m2, ioe, ioo):
    sk = pl.kernel(
        _scatter_body,
        out_type=jax.ShapeDtypeStruct((2, VP, 64), f32),
        mesh=_sc_mesh(),
        compiler_params=_SC_PARAMS,
        scratch_types=(
            [pltpu.VMEM((K // 2, 128), f32)] * 2
            + [pltpu.VMEM((K // 2,), jnp.int32)] * 4
            + [pltpu.VMEM((K // 2, 64), f32)] * 2
            + [pltpu.VMEM((64, 64), f32),
               pltpu.VMEM_SHARED((VP, 64), f32)]
            + [pltpu.SemaphoreType.DMA] * 6
        ),
    )
    return sk(m2, ioe, ioo)


def _deg_body(io, dego, io_0, io_1, ones_v, zd_v, D_sh, si0, si1, ss0, ss1):
    sid = lax.axis_index("s")
    cid = lax.axis_index("c")
    wid = sid * 2 + cid
    iov = (io_0, io_1)
    s_i = (si0, si1)
    s_s = (ss0, ss1)
    _zero_fill(zd_v, 64)

    def orow(r, carry):
        ones_v[r, pl.ds(0, 16)] = jnp.ones((16,), f32)
        return carry

    lax.fori_loop(0, K, orow, 0)
    for i in range(RPT // 64):
        pltpu.sync_copy(zd_v, D_sh.at[pl.ds(sid * RPT + i * 64, 64)])
    plsc.subcore_barrier()

    def base(c):
        return wid * EPW + c * K

    dL = {0: pltpu.async_copy(io.at[pl.ds(base(0), K)], iov[0], s_i[0])}
    dS = {}
    for c in range(NCH):
        j = c & 1
        dL[c].wait()
        dS[c] = pltpu.async_copy(ones_v, D_sh.at[iov[j]], add=True, sem=s_s[j])
        if c + 1 < NCH:
            if c >= 1:
                dS[c - 1].wait()
            dL[c + 1] = pltpu.async_copy(io.at[pl.ds(base(c + 1), K)], iov[1 - j], s_i[1 - j])
    dS[NCH - 2].wait()
    dS[NCH - 1].wait()
    plsc.subcore_barrier()
    pltpu.sync_copy(D_sh.at[pl.ds(sid * RPT, RPT)], dego.at[cid, pl.ds(sid * RPT, RPT)])


def _sc_deg(io):
    dk = pl.kernel(
        _deg_body,
        out_type=jax.ShapeDtypeStruct((2, VP, 16), f32),
        mesh=_sc_mesh(),
        compiler_params=_SC_PARAMS,
        scratch_types=(
            [pltpu.VMEM((K,), jnp.int32)] * 2
            + [pltpu.VMEM((K, 16), f32),
               pltpu.VMEM((64, 16), f32),
               pltpu.VMEM_SHARED((VP, 16), f32)]
            + [pltpu.SemaphoreType.DMA] * 4
        ),
    )
    return dk(io)


def _edge_body(m1_ref, j_ref, w2_ref, b2_ref, wa_ref, wb_ref, o_ref):
    x = (m1_ref[...].astype(f32) + j_ref[:, 0:1] * wa_ref[...]
         + j_ref[:, 1:2] * wb_ref[...])
    x = jnp.maximum(x, 0.0)
    y = lax.dot_general(x, w2_ref[...], (((1,), (0,)), ((), ())),
                        preferred_element_type=f32) + b2_ref[...]
    o_ref[...] = jnp.maximum(y, 0.0)


def _tc_edge(m1raw, j2, W2bd, b2d, wa, wb):
    return pl.pallas_call(
        _edge_body,
        grid=(EH // BEH,),
        in_specs=[
            pl.BlockSpec((BEH, 128), lambda i: (i, 0)),
            pl.BlockSpec((BEH, 2), lambda i: (i, 0)),
            pl.BlockSpec((128, 128), lambda i: (0, 0)),
            pl.BlockSpec((1, 128), lambda i: (0, 0)),
            pl.BlockSpec((1, 128), lambda i: (0, 0)),
            pl.BlockSpec((1, 128), lambda i: (0, 0)),
        ],
        out_specs=pl.BlockSpec((BEH, 128), lambda i: (i, 0)),
        out_shape=jax.ShapeDtypeStruct((EH, 128), f32),
    )(m1raw, j2, W2bd, b2d, wa, wb)


def _node_body(s2_ref, st_ref, dvec_ref, ab_ref, bb_ref, ob_ref, t_ref,
               w3t_ref, wiht_ref, whht_ref, bih_ref, bhh_ref,
               o1st_ref, o2t_ref, ob2_ref, o3tp_ref, w1at_ref, w1bt_ref,
               stn_ref, ap_ref, bp_ref, y_ref, l_ref):
    i = pl.program_id(0)
    s = s2_ref[0] + s2_ref[1]
    msg = lax.dot_general(s, w3t_ref[...], (((1,), (0,)), ((), ())),
                          preferred_element_type=f32) + dvec_ref[...]
    st = st_ref[...]
    gi = lax.dot_general(msg, wiht_ref[...], (((1,), (0,)), ((), ())),
                         preferred_element_type=f32) + bih_ref[...]
    gh = lax.dot_general(st, whht_ref[...], (((1,), (0,)), ((), ())),
                         preferred_element_type=f32) + bhh_ref[...]
    r = jax.nn.sigmoid(gi[:, 0:128] + gh[:, 0:128])
    z = jax.nn.sigmoid(gi[:, 128:256] + gh[:, 128:256])
    n = jnp.tanh(gi[:, 256:384] + r * gh[:, 256:384])
    stn = (1.0 - z) * n + z * st
    stn_ref[...] = stn
    o1 = lax.dot_general(stn, o1st_ref[...], (((1,), (0,)), ((), ())),
                         preferred_element_type=f32) + ob_ref[...]
    o1 = jnp.maximum(o1, 0.0)
    o2 = lax.dot_general(o1, o2t_ref[...], (((1,), (0,)), ((), ())),
                         preferred_element_type=f32) + ob2_ref[...]
    o2 = jnp.maximum(o2, 0.0)
    l01 = lax.dot_general(o2, o3tp_ref[...], (((1,), (0,)), ((), ())),
                          preferred_element_type=f32)
    l0 = l01[:, 0:1]
    l1 = l01[:, 1:2]
    m = jnp.maximum(l0, l1)
    lse = m + jnp.log(jnp.exp(l0 - m) + jnp.exp(l1 - m))
    y_ref[...] = jnp.exp(l0 - lse)
    ll = jnp.concatenate([l0 - lse, l1 - lse], axis=1)
    d = ll - jnp.log(t_ref[...])
    rows = i * BM + lax.broadcasted_iota(jnp.int32, (BM, 2), 0)
    sq = jnp.where(rows < V, d * d, 0.0)
    part = jnp.sum(sq, axis=(0, 1), keepdims=True)

    @pl.when(i == 0)
    def _():
        l_ref[...] = jnp.zeros((1, 1), f32)

    l_ref[...] += part
    ap_ref[...] = (lax.dot_general(stn, w1at_ref[...], (((1,), (0,)), ((), ())),
                                   preferred_element_type=f32)
                   + ab_ref[...]).astype(bf16)
    bp_ref[...] = (lax.dot_general(stn, w1bt_ref[...], (((1,), (0,)), ((), ())),
                                   preferred_element_type=f32)
                   + bb_ref[...]).astype(bf16)


def _tc_node(s2, st, dvec, abias, bbias, obias, tpad, W3T, WihT, WhhT,
             bihr, bhhr, O1sT, O2T, ob2r, O3Tp, W1aT, W1bT):
    return pl.pallas_call(
        _node_body,
        grid=(VP // BM,),
        in_specs=[
            pl.BlockSpec((2, BM, 64), lambda i: (0, i, 0)),
            pl.BlockSpec((BM, 128), lambda i: (i, 0)),
            pl.BlockSpec((BM, 128), lambda i: (i, 0)),
            pl.BlockSpec((BM, 64), lambda i: (i, 0)),
            pl.BlockSpec((BM, 64), lambda i: (i, 0)),
            pl.BlockSpec((BM, 64), lambda i: (i, 0)),
            pl.BlockSpec((BM, 2), lambda i: (i, 0)),
            pl.BlockSpec((64, 128), lambda i: (0, 0)),
            pl.BlockSpec((128, 384), lambda i: (0, 0)),
            pl.BlockSpec((128, 384), lambda i: (0, 0)),
            pl.BlockSpec((1, 384), lambda i: (0, 0)),
            pl.BlockSpec((1, 384), lambda i: (0, 0)),
            pl.BlockSpec((128, 64), lambda i: (0, 0)),
            pl.BlockSpec((64, 64), lambda i: (0, 0)),
            pl.BlockSpec((1, 64), lambda i: (0, 0)),
            pl.BlockSpec((64, 128), lambda i: (0, 0)),
            pl.BlockSpec((128, 64), lambda i: (0, 0)),
            pl.BlockSpec((128, 64), lambda i: (0, 0)),
        ],
        out_specs=[
            pl.BlockSpec((BM, 128), lambda i: (i, 0)),
            pl.BlockSpec((BM, 64), lambda i: (i, 0)),
            pl.BlockSpec((BM, 64), lambda i: (i, 0)),
            pl.BlockSpec((BM, 1), lambda i: (i, 0)),
            pl.BlockSpec((1, 1), lambda i: (0, 0)),
        ],
        out_shape=[
            jax.ShapeDtypeStruct((VP, 128), f32),
            jax.ShapeDtypeStruct((VP, 64), bf16),
            jax.ShapeDtypeStruct((VP, 64), bf16),
            jax.ShapeDtypeStruct((VP, 1), f32),
            jax.ShapeDtypeStruct((1, 1), f32),
        ],
    )(s2, st, dvec, abias, bbias, obias, tpad, W3T, WihT, WhhT,
      bihr, bhhr, O1sT, O2T, ob2r, O3Tp, W1aT, W1bT)


def kernel(J_msg, b, msg_node, idx_msg_edge, target, W1, b1, W2, b2, W3, b3,
           Wih, Whh, bih, bhh, O1, ob1, O2, ob2, O3, ob3):
    del idx_msg_edge
    W1aT = W1[:, 0:128].T
    W1bT = W1[:, 132:260].T
    u = W1[:, 128] - W1[:, 129]
    v = W1[:, 261] - W1[:, 260]
    w = (W1[:, 130] - W1[:, 131]) + (W1[:, 263] - W1[:, 262])
    bp = jnp.pad(b, ((0, VP - V), (0, 0)))
    abias = bp * u[None, :]
    bbias = bp * v[None, :] + b1[None, :]
    obias = bp * (O1[:, 128] - O1[:, 129])[None, :] + ob1[None, :]
    tpad = jnp.pad(target, ((0, VP - V), (0, 0)), constant_values=1.0)
    W2T = W2.T
    W2bd = jnp.zeros((128, 128), f32).at[:64, :64].set(W2T).at[64:, 64:].set(W2T)
    b2d = jnp.concatenate([b2, b2])[None, :]
    z64 = jnp.zeros((64,), f32)
    W3T = W3.T
    WihT = Wih.T
    WhhT = Whh.T
    O1sT = O1[:, 0:128].T
    O2T = O2.T
    O3Tp = jnp.pad(O3.T, ((0, 0), (0, 128 - 2)))
    bihr = bih[None, :]
    bhhr = bhh[None, :]
    ob2r = ob2[None, :]
    wa = jnp.concatenate([w, z64])[None, :]
    wb = jnp.concatenate([z64, w])[None, :]
    ii = msg_node[:, 0].astype(jnp.int32)
    io = msg_node[:, 1].astype(jnp.int32)
    j2 = J_msg.reshape(EH, 2)
    ioe = io[0::2]
    ioo = io[1::2]

    state = jnp.zeros((VP, H), f32)
    ap = abias.astype(bf16)
    bpp = bbias.astype(bf16)
    dvec = None
    ys = []
    lsum = None
    for t in range(NPROP):
        m1raw = _sc_gather(ap, bpp, ii, io)
        m2 = _tc_edge(m1raw, j2, W2bd, b2d, wa, wb)
        if t == 0:
            deg2 = _sc_deg(io)
            deg = deg2[0, :, 0] + deg2[1, :, 0]
            dvec = deg[:, None] * b3[None, :]
        s2 = _sc_scatter(m2, ioe, ioo)
        state, ap, bpp, y, lsum = _tc_node(
            s2, state, dvec, abias, bbias, obias, tpad, W3T, WihT, WhhT,
            bihr, bhhr, O1sT, O2T, ob2r, O3Tp, W1aT, W1bT)
        ys.append(y)
    y_step = jnp.concatenate(ys, axis=1)[:V, :]
    loss = (lsum[0, 0] / jnp.float32(V)).astype(f32)
    return (y_step, loss)


# speedup vs baseline: 1.1820x; 1.1820x over previous
import functools

import jax
import jax.numpy as jnp
from jax import lax
from jax.experimental import pallas as pl
from jax.experimental.pallas import tpu as pltpu
from jax.experimental.pallas import tpu_sc as plsc

V = 10000
E = 320000
H = 128
NPROP = 5
VP = 10240
NW = 32
EPW = E // NW
K = 400
NCH = EPW // K
RPT = VP // 16
BM = 1024
EH = E // 2
BEH = 1000

f32 = jnp.float32
bf16 = jnp.bfloat16


def _sc_mesh():
    return plsc.VectorSubcoreMesh(core_axis_name="c", subcore_axis_name="s")


_SC_PARAMS = pltpu.CompilerParams(use_tc_tiling_on_sc=False, needs_layout_passes=False)


KG = 200
NCHG = EPW // KG


def _gather_body(ap, bp, ii, io, m1,
                 ii_0, ii_1, io_0, io_1, a_0, a_1, b_0, b_1, o_0, o_1,
                 sii0, sii1, sio0, sio1, sa0, sa1, sb0, sb1, so0, so1):
    wid = lax.axis_index("s") * 2 + lax.axis_index("c")
    iiv = (ii_0, ii_1)
    iov = (io_0, io_1)
    av = (a_0, a_1)
    bv = (b_0, b_1)
    ov = (o_0, o_1)
    s_ii = (sii0, sii1)
    s_io = (sio0, sio1)
    s_a = (sa0, sa1)
    s_b = (sb0, sb1)
    s_o = (so0, so1)

    def base(c):
        return wid * EPW + c * KG

    dI = {}
    dG = {}
    dO = {}

    def issue_idx(c):
        j = c & 1
        dI[c] = (pltpu.async_copy(ii.at[pl.ds(base(c), KG)], iiv[j], s_ii[j]),
                 pltpu.async_copy(io.at[pl.ds(base(c), KG)], iov[j], s_io[j]))

    def issue_g(c):
        j = c & 1
        dG[c] = (pltpu.async_copy(ap.at[iiv[j]], av[j], s_a[j]),
                 pltpu.async_copy(bp.at[iov[j]], bv[j], s_b[j]))

    issue_idx(0)
    dI[0][0].wait()
    dI[0][1].wait()
    issue_g(0)
    if NCHG > 1:
        issue_idx(1)
    for c in range(NCHG):
        j = c & 1
        dG[c][0].wait()
        dG[c][1].wait()
        if c + 2 < NCHG:
            issue_idx(c + 2)
        if c + 1 < NCHG:
            dI[c + 1][0].wait()
            dI[c + 1][1].wait()
            issue_g(c + 1)
        jo = (c // 2) & 1
        roff = (c % 2) * (KG // 2)
        if c % 2 == 0 and c // 2 >= 2:
            dO[c // 2 - 2].wait()
        a_r, b_r, o_r = av[j], bv[j], ov[jo]

        def row(r, carry):
            for cc in range(4):
                sr = 2 * r + (1 if cc >= 2 else 0)
                g = cc % 2
                s = a_r[sr, pl.ds(32 * g, 32)] + b_r[sr, pl.ds(32 * g, 32)]
                lo, hi = plsc.unpack(s, format=plsc.PackFormat.INTERLEAVED)
                col = 64 * (1 if cc >= 2 else 0) + 32 * g
                o_r[roff + r, pl.ds(col, 16)] = lo
                o_r[roff + r, pl.ds(col + 16, 16)] = hi
            return carry

        lax.fori_loop(0, KG // 2, row, 0, unroll=2)
        if c % 2 == 1:
            dO[c // 2] = pltpu.async_copy(
                ov[jo], m1.at[pl.ds(base(c - 1) // 2, KG)], s_o[jo])
    dO[NCHG // 2 - 2].wait()
    dO[NCHG // 2 - 1].wait()


def _sc_gather(ap, bp, ii, io):
    gk = pl.kernel(
        _gather_body,
        out_type=jax.ShapeDtypeStruct((EH, 128), f32),
        mesh=_sc_mesh(),
        compiler_params=_SC_PARAMS,
        scratch_types=(
            [pltpu.VMEM((KG,), jnp.int32)] * 4
            + [pltpu.VMEM((KG, 64), bf16)] * 4
            + [pltpu.VMEM((KG, 128), f32)] * 2
            + [pltpu.SemaphoreType.DMA] * 10
        ),
    )
    return gk(ap, bp, ii, io)


def _zero_fill(buf, rows):
    def zrow(r, carry):
        for cc in range(buf.shape[1] // 16):
            buf[r, pl.ds(16 * cc, 16)] = jnp.zeros((16,), f32)
        return carry

    lax.fori_loop(0, rows, zrow, 0)


def _scatter_body(m2, ioe, ioo, s2o, m2_0, m2_1, ioe_0, ioe_1, ioo_0, ioo_1,
                  se_v, so_v, z_v, S_sh, sm0, sm1, se0, se1, so0, so1):
    sid = lax.axis_index("s")
    cid = lax.axis_index("c")
    wid = sid * 2 + cid
    m2v = (m2_0, m2_1)
    ioev = (ioe_0, ioe_1)
    ioov = (ioo_0, ioo_1)
    s_m = (sm0, sm1)
    s_e = (se0, se1)
    s_o = (so0, so1)
    _zero_fill(z_v, 64)
    for i in range(RPT // 64):
        pltpu.sync_copy(z_v, S_sh.at[pl.ds(sid * RPT + i * 64, 64)])
    plsc.subcore_barrier()

    def base(c):
        return wid * EPW + c * K

    dL = {}

    def load(c):
        j = c & 1
        hb = pl.multiple_of(base(c) // 2, 8)
        dL[c] = (pltpu.async_copy(m2.at[pl.ds(hb, K // 2)], m2v[j], s_m[j]),
                 pltpu.async_copy(ioe.at[pl.ds(hb, K // 2)], ioev[j], s_e[j]),
                 pltpu.async_copy(ioo.at[pl.ds(hb, K // 2)], ioov[j], s_o[j]))

    load(0)
    for c in range(NCH):
        j = c & 1
        for d in dL[c]:
            d.wait()
        if c + 1 < NCH:
            load(c + 1)
        m_r = m2v[j]

        def srow(r, carry):
            for cc in range(4):
                sl = pl.ds(16 * cc, 16)
                se_v[r, sl] = m_r[r, sl]
                so_v[r, sl] = m_r[r, pl.ds(64 + 16 * cc, 16)]
            return carry

        lax.fori_loop(0, K // 2, srow, 0, unroll=2)
        pltpu.sync_copy(se_v, S_sh.at[ioev[j]], add=True)
        pltpu.sync_copy(so_v, S_sh.at[ioov[j]], add=True)
    plsc.subcore_barrier()
    pltpu.sync_copy(S_sh.at[pl.ds(sid * RPT, RPT)], s2o.at[cid, pl.ds(sid * RPT, RPT)])


def _sc_scatter(m2, ioe, ioo):
    sk = pl.kernel(
        _scatter_body,
        out_type=jax.ShapeDtypeStruct((2, VP, 64), f32),
        mesh=_sc_mesh(),
        compiler_params=_SC_PARAMS,
        scratch_types=(
            [pltpu.VMEM((K // 2, 128), f32)] * 2
            + [pltpu.VMEM((K // 2,), jnp.int32)] * 4
            + [pltpu.VMEM((K // 2, 64), f32)] * 2
            + [pltpu.VMEM((64, 64), f32),
               pltpu.VMEM_SHARED((VP, 64), f32)]
            + [pltpu.SemaphoreType.DMA] * 6
        ),
    )
    return sk(m2, ioe, ioo)


def _deg_body(io, dego, io_0, io_1, ones_v, zd_v, D_sh, si0, si1, ss0, ss1):
    sid = lax.axis_index("s")
    cid = lax.axis_index("c")
    wid = sid * 2 + cid
    iov = (io_0, io_1)
    s_i = (si0, si1)
    s_s = (ss0, ss1)
    _zero_fill(zd_v, 64)

    def orow(r, carry):
        ones_v[r, pl.ds(0, 16)] = jnp.ones((16,), f32)
        return carry

    lax.fori_loop(0, K, orow, 0)
    for i in range(RPT // 64):
        pltpu.sync_copy(zd_v, D_sh.at[pl.ds(sid * RPT + i * 64, 64)])
    plsc.subcore_barrier()

    def base(c):
        return wid * EPW + c * K

    dL = {0: pltpu.async_copy(io.at[pl.ds(base(0), K)], iov[0], s_i[0])}
    dS = {}
    for c in range(NCH):
        j = c & 1
        dL[c].wait()
        dS[c] = pltpu.async_copy(ones_v, D_sh.at[iov[j]], add=True, sem=s_s[j])
        if c + 1 < NCH:
            if c >= 1:
                dS[c - 1].wait()
            dL[c + 1] = pltpu.async_copy(io.at[pl.ds(base(c + 1), K)], iov[1 - j], s_i[1 - j])
    dS[NCH - 2].wait()
    dS[NCH - 1].wait()
    plsc.subcore_barrier()
    pltpu.sync_copy(D_sh.at[pl.ds(sid * RPT, RPT)], dego.at[cid, pl.ds(sid * RPT, RPT)])


def _sc_deg(io):
    dk = pl.kernel(
        _deg_body,
        out_type=jax.ShapeDtypeStruct((2, VP, 16), f32),
        mesh=_sc_mesh(),
        compiler_params=_SC_PARAMS,
        scratch_types=(
            [pltpu.VMEM((K,), jnp.int32)] * 2
            + [pltpu.VMEM((K, 16), f32),
               pltpu.VMEM((64, 16), f32),
               pltpu.VMEM_SHARED((VP, 16), f32)]
            + [pltpu.SemaphoreType.DMA] * 4
        ),
    )
    return dk(io)


def _edge_body(m1_ref, j_ref, w2_ref, b2_ref, wa_ref, wb_ref, o_ref):
    x = (m1_ref[...] + j_ref[:, 0:1] * wa_ref[...]
         + j_ref[:, 1:2] * wb_ref[...])
    x = jnp.maximum(x, 0.0)
    y = lax.dot_general(x, w2_ref[...], (((1,), (0,)), ((), ())),
                        preferred_element_type=f32) + b2_ref[...]
    o_ref[...] = jnp.maximum(y, 0.0)


def _tc_edge(m1raw, j2, W2bd, b2d, wa, wb):
    return pl.pallas_call(
        _edge_body,
        grid=(EH // BEH,),
        in_specs=[
            pl.BlockSpec((BEH, 128), lambda i: (i, 0)),
            pl.BlockSpec((BEH, 2), lambda i: (i, 0)),
            pl.BlockSpec((128, 128), lambda i: (0, 0)),
            pl.BlockSpec((1, 128), lambda i: (0, 0)),
            pl.BlockSpec((1, 128), lambda i: (0, 0)),
            pl.BlockSpec((1, 128), lambda i: (0, 0)),
        ],
        out_specs=pl.BlockSpec((BEH, 128), lambda i: (i, 0)),
        out_shape=jax.ShapeDtypeStruct((EH, 128), f32),
    )(m1raw, j2, W2bd, b2d, wa, wb)


def _node_body(s2_ref, st_ref, dvec_ref, ab_ref, bb_ref, ob_ref, t_ref,
               w3t_ref, wiht_ref, whht_ref, bih_ref, bhh_ref,
               o1st_ref, o2t_ref, ob2_ref, o3tp_ref, w1at_ref, w1bt_ref,
               stn_ref, ap_ref, bp_ref, y_ref, l_ref):
    i = pl.program_id(0)
    s = s2_ref[0] + s2_ref[1]
    msg = lax.dot_general(s, w3t_ref[...], (((1,), (0,)), ((), ())),
                          preferred_element_type=f32) + dvec_ref[...]
    st = st_ref[...]
    gi = lax.dot_general(msg, wiht_ref[...], (((1,), (0,)), ((), ())),
                         preferred_element_type=f32) + bih_ref[...]
    gh = lax.dot_general(st, whht_ref[...], (((1,), (0,)), ((), ())),
                         preferred_element_type=f32) + bhh_ref[...]
    r = jax.nn.sigmoid(gi[:, 0:128] + gh[:, 0:128])
    z = jax.nn.sigmoid(gi[:, 128:256] + gh[:, 128:256])
    n = jnp.tanh(gi[:, 256:384] + r * gh[:, 256:384])
    stn = (1.0 - z) * n + z * st
    stn_ref[...] = stn
    o1 = lax.dot_general(stn, o1st_ref[...], (((1,), (0,)), ((), ())),
                         preferred_element_type=f32) + ob_ref[...]
    o1 = jnp.maximum(o1, 0.0)
    o2 = lax.dot_general(o1, o2t_ref[...], (((1,), (0,)), ((), ())),
                         preferred_element_type=f32) + ob2_ref[...]
    o2 = jnp.maximum(o2, 0.0)
    l01 = lax.dot_general(o2, o3tp_ref[...], (((1,), (0,)), ((), ())),
                          preferred_element_type=f32)
    l0 = l01[:, 0:1]
    l1 = l01[:, 1:2]
    m = jnp.maximum(l0, l1)
    lse = m + jnp.log(jnp.exp(l0 - m) + jnp.exp(l1 - m))
    y_ref[...] = jnp.exp(l0 - lse)
    ll = jnp.concatenate([l0 - lse, l1 - lse], axis=1)
    d = ll - jnp.log(t_ref[...])
    rows = i * BM + lax.broadcasted_iota(jnp.int32, (BM, 2), 0)
    sq = jnp.where(rows < V, d * d, 0.0)
    part = jnp.sum(sq, axis=(0, 1), keepdims=True)

    @pl.when(i == 0)
    def _():
        l_ref[...] = jnp.zeros((1, 1), f32)

    l_ref[...] += part
    ap_ref[...] = (lax.dot_general(stn, w1at_ref[...], (((1,), (0,)), ((), ())),
                                   preferred_element_type=f32)
                   + ab_ref[...]).astype(bf16)
    bp_ref[...] = (lax.dot_general(stn, w1bt_ref[...], (((1,), (0,)), ((), ())),
                                   preferred_element_type=f32)
                   + bb_ref[...]).astype(bf16)


def _tc_node(s2, st, dvec, abias, bbias, obias, tpad, W3T, WihT, WhhT,
             bihr, bhhr, O1sT, O2T, ob2r, O3Tp, W1aT, W1bT):
    return pl.pallas_call(
        _node_body,
        grid=(VP // BM,),
        in_specs=[
            pl.BlockSpec((2, BM, 64), lambda i: (0, i, 0)),
            pl.BlockSpec((BM, 128), lambda i: (i, 0)),
            pl.BlockSpec((BM, 128), lambda i: (i, 0)),
            pl.BlockSpec((BM, 64), lambda i: (i, 0)),
            pl.BlockSpec((BM, 64), lambda i: (i, 0)),
            pl.BlockSpec((BM, 64), lambda i: (i, 0)),
            pl.BlockSpec((BM, 2), lambda i: (i, 0)),
            pl.BlockSpec((64, 128), lambda i: (0, 0)),
            pl.BlockSpec((128, 384), lambda i: (0, 0)),
            pl.BlockSpec((128, 384), lambda i: (0, 0)),
            pl.BlockSpec((1, 384), lambda i: (0, 0)),
            pl.BlockSpec((1, 384), lambda i: (0, 0)),
            pl.BlockSpec((128, 64), lambda i: (0, 0)),
            pl.BlockSpec((64, 64), lambda i: (0, 0)),
            pl.BlockSpec((1, 64), lambda i: (0, 0)),
            pl.BlockSpec((64, 128), lambda i: (0, 0)),
            pl.BlockSpec((128, 64), lambda i: (0, 0)),
            pl.BlockSpec((128, 64), lambda i: (0, 0)),
        ],
        out_specs=[
            pl.BlockSpec((BM, 128), lambda i: (i, 0)),
            pl.BlockSpec((BM, 64), lambda i: (i, 0)),
            pl.BlockSpec((BM, 64), lambda i: (i, 0)),
            pl.BlockSpec((BM, 1), lambda i: (i, 0)),
            pl.BlockSpec((1, 1), lambda i: (0, 0)),
        ],
        out_shape=[
            jax.ShapeDtypeStruct((VP, 128), f32),
            jax.ShapeDtypeStruct((VP, 64), bf16),
            jax.ShapeDtypeStruct((VP, 64), bf16),
            jax.ShapeDtypeStruct((VP, 1), f32),
            jax.ShapeDtypeStruct((1, 1), f32),
        ],
    )(s2, st, dvec, abias, bbias, obias, tpad, W3T, WihT, WhhT,
      bihr, bhhr, O1sT, O2T, ob2r, O3Tp, W1aT, W1bT)


def kernel(J_msg, b, msg_node, idx_msg_edge, target, W1, b1, W2, b2, W3, b3,
           Wih, Whh, bih, bhh, O1, ob1, O2, ob2, O3, ob3):
    del idx_msg_edge
    perm = jnp.array([32 * g + x for g in range(2)
                      for k in range(16) for x in (k, k + 16)], dtype=jnp.int32)
    W1aT = W1[:, 0:128].T[:, perm]
    W1bT = W1[:, 132:260].T[:, perm]
    u = W1[:, 128] - W1[:, 129]
    v = W1[:, 261] - W1[:, 260]
    w = (W1[:, 130] - W1[:, 131]) + (W1[:, 263] - W1[:, 262])
    bp = jnp.pad(b, ((0, VP - V), (0, 0)))
    abias = (bp * u[None, :])[:, perm]
    bbias = (bp * v[None, :] + b1[None, :])[:, perm]
    obias = bp * (O1[:, 128] - O1[:, 129])[None, :] + ob1[None, :]
    tpad = jnp.pad(target, ((0, VP - V), (0, 0)), constant_values=1.0)
    W2T = W2.T
    W2bd = jnp.zeros((128, 128), f32).at[:64, :64].set(W2T).at[64:, 64:].set(W2T)
    b2d = jnp.concatenate([b2, b2])[None, :]
    z64 = jnp.zeros((64,), f32)
    W3T = W3.T
    WihT = Wih.T
    WhhT = Whh.T
    O1sT = O1[:, 0:128].T
    O2T = O2.T
    O3Tp = jnp.pad(O3.T, ((0, 0), (0, 128 - 2)))
    bihr = bih[None, :]
    bhhr = bhh[None, :]
    ob2r = ob2[None, :]
    wa = jnp.concatenate([w, z64])[None, :]
    wb = jnp.concatenate([z64, w])[None, :]
    ii = msg_node[:, 0].astype(jnp.int32)
    io = msg_node[:, 1].astype(jnp.int32)
    j2 = J_msg.reshape(EH, 2)
    ioe = io[0::2]
    ioo = io[1::2]

    state = jnp.zeros((VP, H), f32)
    ap = abias.astype(bf16)
    bpp = bbias.astype(bf16)
    dvec = None
    ys = []
    lsum = None
    for t in range(NPROP):
        m1raw = _sc_gather(ap, bpp, ii, io)
        m2 = _tc_edge(m1raw, j2, W2bd, b2d, wa, wb)
        if t == 0:
            deg2 = _sc_deg(io)
            deg = deg2[0, :, 0] + deg2[1, :, 0]
            dvec = deg[:, None] * b3[None, :]
        s2 = _sc_scatter(m2, ioe, ioo)
        state, ap, bpp, y, lsum = _tc_node(
            s2, state, dvec, abias, bbias, obias, tpad, W3T, WihT, WhhT,
            bihr, bhhr, O1sT, O2T, ob2r, O3Tp, W1aT, W1bT)
        ys.append(y)
    y_step = jnp.concatenate(ys, axis=1)[:V, :]
    loss = (lsum[0, 0] / jnp.float32(V)).astype(f32)
    return (y_step, loss)


# speedup vs baseline: 1.2934x; 1.0943x over previous
import functools

import jax
import jax.numpy as jnp
from jax import lax
from jax.experimental import pallas as pl
from jax.experimental.pallas import tpu as pltpu
from jax.experimental.pallas import tpu_sc as plsc

V = 10000
E = 320000
H = 128
NPROP = 5
VP = 10240
NW = 32
EPW = E // NW
K = 400
NCH = EPW // K
RPT = VP // 16
BM = 1024
EH = E // 2
BEH = 2000

f32 = jnp.float32
bf16 = jnp.bfloat16


def _sc_mesh():
    return plsc.VectorSubcoreMesh(core_axis_name="c", subcore_axis_name="s")


_SC_PARAMS = pltpu.CompilerParams(use_tc_tiling_on_sc=False, needs_layout_passes=False)


KG = 200
NCHG = EPW // KG


def _gather_body(ap, bp, ii, io, m1,
                 ii_0, ii_1, io_0, io_1, a_0, a_1, b_0, b_1, o_0, o_1,
                 sii0, sii1, sio0, sio1, sa0, sa1, sb0, sb1, so0, so1):
    wid = lax.axis_index("s") * 2 + lax.axis_index("c")
    iiv = (ii_0, ii_1)
    iov = (io_0, io_1)
    av = (a_0, a_1)
    bv = (b_0, b_1)
    ov = (o_0, o_1)
    s_ii = (sii0, sii1)
    s_io = (sio0, sio1)
    s_a = (sa0, sa1)
    s_b = (sb0, sb1)
    s_o = (so0, so1)

    def base(c):
        return wid * EPW + c * KG

    dI = {}
    dG = {}
    dO = {}

    def issue_idx(c):
        j = c & 1
        dI[c] = (pltpu.async_copy(ii.at[pl.ds(base(c), KG)], iiv[j], s_ii[j]),
                 pltpu.async_copy(io.at[pl.ds(base(c), KG)], iov[j], s_io[j]))

    def issue_g(c):
        j = c & 1
        dG[c] = (pltpu.async_copy(ap.at[iiv[j]], av[j], s_a[j]),
                 pltpu.async_copy(bp.at[iov[j]], bv[j], s_b[j]))

    issue_idx(0)
    dI[0][0].wait()
    dI[0][1].wait()
    issue_g(0)
    if NCHG > 1:
        issue_idx(1)
    for c in range(NCHG):
        j = c & 1
        dG[c][0].wait()
        dG[c][1].wait()
        if c + 2 < NCHG:
            issue_idx(c + 2)
        if c + 1 < NCHG:
            dI[c + 1][0].wait()
            dI[c + 1][1].wait()
            issue_g(c + 1)
        jo = (c // 2) & 1
        roff = (c % 2) * (KG // 2)
        if c % 2 == 0 and c // 2 >= 2:
            dO[c // 2 - 2].wait()
        a_r, b_r, o_r = av[j], bv[j], ov[jo]

        def row(r, carry):
            for cc in range(4):
                sr = 2 * r + (1 if cc >= 2 else 0)
                g = cc % 2
                s = a_r[sr, pl.ds(32 * g, 32)] + b_r[sr, pl.ds(32 * g, 32)]
                lo, hi = plsc.unpack(s, format=plsc.PackFormat.INTERLEAVED)
                col = 64 * (1 if cc >= 2 else 0) + 32 * g
                o_r[roff + r, pl.ds(col, 16)] = lo
                o_r[roff + r, pl.ds(col + 16, 16)] = hi
            return carry

        lax.fori_loop(0, KG // 2, row, 0, unroll=2)
        if c % 2 == 1:
            dO[c // 2] = pltpu.async_copy(
                ov[jo], m1.at[pl.ds(base(c - 1) // 2, KG)], s_o[jo])
    dO[NCHG // 2 - 2].wait()
    dO[NCHG // 2 - 1].wait()


def _sc_gather(ap, bp, ii, io):
    gk = pl.kernel(
        _gather_body,
        out_type=jax.ShapeDtypeStruct((EH, 128), f32),
        mesh=_sc_mesh(),
        compiler_params=_SC_PARAMS,
        scratch_types=(
            [pltpu.VMEM((KG,), jnp.int32)] * 4
            + [pltpu.VMEM((KG, 64), bf16)] * 4
            + [pltpu.VMEM((KG, 128), f32)] * 2
            + [pltpu.SemaphoreType.DMA] * 10
        ),
    )
    return gk(ap, bp, ii, io)


def _zero_fill(buf, rows):
    def zrow(r, carry):
        for cc in range(buf.shape[1] // 16):
            buf[r, pl.ds(16 * cc, 16)] = jnp.zeros((16,), f32)
        return carry

    lax.fori_loop(0, rows, zrow, 0)


def _scatter_body(m2, ioe, ioo, s2o, m2_0, m2_1, ioe_0, ioe_1, ioo_0, ioo_1,
                  se_v, so_v, z_v, S_sh, sm0, sm1, se0, se1, so0, so1):
    sid = lax.axis_index("s")
    cid = lax.axis_index("c")
    wid = sid * 2 + cid
    m2v = (m2_0, m2_1)
    ioev = (ioe_0, ioe_1)
    ioov = (ioo_0, ioo_1)
    s_m = (sm0, sm1)
    s_e = (se0, se1)
    s_o = (so0, so1)
    _zero_fill(z_v, 64)
    for i in range(RPT // 64):
        pltpu.sync_copy(z_v, S_sh.at[pl.ds(sid * RPT + i * 64, 64)])
    plsc.subcore_barrier()

    def base(c):
        return wid * EPW + c * K

    dL = {}

    def load(c):
        j = c & 1
        hb = pl.multiple_of(base(c) // 2, 8)
        dL[c] = (pltpu.async_copy(m2.at[pl.ds(hb, K // 2)], m2v[j], s_m[j]),
                 pltpu.async_copy(ioe.at[pl.ds(hb, K // 2)], ioev[j], s_e[j]),
                 pltpu.async_copy(ioo.at[pl.ds(hb, K // 2)], ioov[j], s_o[j]))

    load(0)
    for c in range(NCH):
        j = c & 1
        for d in dL[c]:
            d.wait()
        if c + 1 < NCH:
            load(c + 1)
        m_r = m2v[j]

        def srow(r, carry):
            for cc in range(4):
                sl = pl.ds(16 * cc, 16)
                se_v[r, sl] = m_r[r, sl]
                so_v[r, sl] = m_r[r, pl.ds(64 + 16 * cc, 16)]
            return carry

        lax.fori_loop(0, K // 2, srow, 0, unroll=2)
        pltpu.sync_copy(se_v, S_sh.at[ioev[j]], add=True)
        pltpu.sync_copy(so_v, S_sh.at[ioov[j]], add=True)
    plsc.subcore_barrier()
    pltpu.sync_copy(S_sh.at[pl.ds(sid * RPT, RPT)], s2o.at[cid, pl.ds(sid * RPT, RPT)])


def _sc_scatter(m2, ioe, ioo):
    sk = pl.kernel(
        _scatter_body,
        out_type=jax.ShapeDtypeStruct((2, VP, 64), f32),
        mesh=_sc_mesh(),
        compiler_params=_SC_PARAMS,
        scratch_types=(
            [pltpu.VMEM((K // 2, 128), f32)] * 2
            + [pltpu.VMEM((K // 2,), jnp.int32)] * 4
            + [pltpu.VMEM((K // 2, 64), f32)] * 2
            + [pltpu.VMEM((64, 64), f32),
               pltpu.VMEM_SHARED((VP, 64), f32)]
            + [pltpu.SemaphoreType.DMA] * 6
        ),
    )
    return sk(m2, ioe, ioo)


def _deg_body(io, dego, io_0, io_1, ones_v, zd_v, D_sh, si0, si1, ss0, ss1):
    sid = lax.axis_index("s")
    cid = lax.axis_index("c")
    wid = sid * 2 + cid
    iov = (io_0, io_1)
    s_i = (si0, si1)
    s_s = (ss0, ss1)
    _zero_fill(zd_v, 64)

    def orow(r, carry):
        ones_v[r, pl.ds(0, 16)] = jnp.ones((16,), f32)
        return carry

    lax.fori_loop(0, K, orow, 0)
    for i in range(RPT // 64):
        pltpu.sync_copy(zd_v, D_sh.at[pl.ds(sid * RPT + i * 64, 64)])
    plsc.subcore_barrier()

    def base(c):
        return wid * EPW + c * K

    dL = {0: pltpu.async_copy(io.at[pl.ds(base(0), K)], iov[0], s_i[0])}
    dS = {}
    for c in range(NCH):
        j = c & 1
        dL[c].wait()
        dS[c] = pltpu.async_copy(ones_v, D_sh.at[iov[j]], add=True, sem=s_s[j])
        if c + 1 < NCH:
            if c >= 1:
                dS[c - 1].wait()
            dL[c + 1] = pltpu.async_copy(io.at[pl.ds(base(c + 1), K)], iov[1 - j], s_i[1 - j])
    dS[NCH - 2].wait()
    dS[NCH - 1].wait()
    plsc.subcore_barrier()
    pltpu.sync_copy(D_sh.at[pl.ds(sid * RPT, RPT)], dego.at[cid, pl.ds(sid * RPT, RPT)])


def _sc_deg(io):
    dk = pl.kernel(
        _deg_body,
        out_type=jax.ShapeDtypeStruct((2, VP, 16), f32),
        mesh=_sc_mesh(),
        compiler_params=_SC_PARAMS,
        scratch_types=(
            [pltpu.VMEM((K,), jnp.int32)] * 2
            + [pltpu.VMEM((K, 16), f32),
               pltpu.VMEM((64, 16), f32),
               pltpu.VMEM_SHARED((VP, 16), f32)]
            + [pltpu.SemaphoreType.DMA] * 4
        ),
    )
    return dk(io)


def _edge_body(m1_ref, j_ref, w2_ref, b2_ref, wa_ref, wb_ref, o_ref):
    x = (m1_ref[...] + j_ref[:, 0:1] * wa_ref[...]
         + j_ref[:, 1:2] * wb_ref[...])
    x = jnp.maximum(x, 0.0)
    y = lax.dot_general(x, w2_ref[...], (((1,), (0,)), ((), ())),
                        preferred_element_type=f32) + b2_ref[...]
    o_ref[...] = jnp.maximum(y, 0.0)


def _tc_edge(m1raw, j2, W2bd, b2d, wa, wb):
    return pl.pallas_call(
        _edge_body,
        grid=(EH // BEH,),
        in_specs=[
            pl.BlockSpec((BEH, 128), lambda i: (i, 0)),
            pl.BlockSpec((BEH, 2), lambda i: (i, 0)),
            pl.BlockSpec((128, 128), lambda i: (0, 0)),
            pl.BlockSpec((1, 128), lambda i: (0, 0)),
            pl.BlockSpec((1, 128), lambda i: (0, 0)),
            pl.BlockSpec((1, 128), lambda i: (0, 0)),
        ],
        out_specs=pl.BlockSpec((BEH, 128), lambda i: (i, 0)),
        out_shape=jax.ShapeDtypeStruct((EH, 128), f32),
    )(m1raw, j2, W2bd, b2d, wa, wb)


def _node_body(s2_ref, st_ref, dvec_ref, ab_ref, bb_ref, ob_ref, t_ref,
               w3t_ref, wiht_ref, whht_ref, bih_ref, bhh_ref,
               o1st_ref, o2t_ref, ob2_ref, o3tp_ref, w1at_ref, w1bt_ref,
               stn_ref, ap_ref, bp_ref, y_ref, l_ref):
    i = pl.program_id(0)
    s = s2_ref[0] + s2_ref[1]
    msg = lax.dot_general(s, w3t_ref[...], (((1,), (0,)), ((), ())),
                          preferred_element_type=f32) + dvec_ref[...]
    st = st_ref[...]
    gi = lax.dot_general(msg, wiht_ref[...], (((1,), (0,)), ((), ())),
                         preferred_element_type=f32) + bih_ref[...]
    gh = lax.dot_general(st, whht_ref[...], (((1,), (0,)), ((), ())),
                         preferred_element_type=f32) + bhh_ref[...]
    r = jax.nn.sigmoid(gi[:, 0:128] + gh[:, 0:128])
    z = jax.nn.sigmoid(gi[:, 128:256] + gh[:, 128:256])
    n = jnp.tanh(gi[:, 256:384] + r * gh[:, 256:384])
    stn = (1.0 - z) * n + z * st
    stn_ref[...] = stn
    o1 = lax.dot_general(stn, o1st_ref[...], (((1,), (0,)), ((), ())),
                         preferred_element_type=f32) + ob_ref[...]
    o1 = jnp.maximum(o1, 0.0)
    o2 = lax.dot_general(o1, o2t_ref[...], (((1,), (0,)), ((), ())),
                         preferred_element_type=f32) + ob2_ref[...]
    o2 = jnp.maximum(o2, 0.0)
    l01 = lax.dot_general(o2, o3tp_ref[...], (((1,), (0,)), ((), ())),
                          preferred_element_type=f32)
    l0 = l01[:, 0:1]
    l1 = l01[:, 1:2]
    m = jnp.maximum(l0, l1)
    lse = m + jnp.log(jnp.exp(l0 - m) + jnp.exp(l1 - m))
    y_ref[...] = jnp.exp(l0 - lse)
    ll = jnp.concatenate([l0 - lse, l1 - lse], axis=1)
    d = ll - jnp.log(t_ref[...])
    rows = i * BM + lax.broadcasted_iota(jnp.int32, (BM, 2), 0)
    sq = jnp.where(rows < V, d * d, 0.0)
    part = jnp.sum(sq, axis=(0, 1), keepdims=True)

    @pl.when(i == 0)
    def _():
        l_ref[...] = jnp.zeros((1, 1), f32)

    l_ref[...] += part
    ap_ref[...] = (lax.dot_general(stn, w1at_ref[...], (((1,), (0,)), ((), ())),
                                   preferred_element_type=f32)
                   + ab_ref[...]).astype(bf16)
    bp_ref[...] = (lax.dot_general(stn, w1bt_ref[...], (((1,), (0,)), ((), ())),
                                   preferred_element_type=f32)
                   + bb_ref[...]).astype(bf16)


def _tc_node(s2, st, dvec, abias, bbias, obias, tpad, W3T, WihT, WhhT,
             bihr, bhhr, O1sT, O2T, ob2r, O3Tp, W1aT, W1bT):
    return pl.pallas_call(
        _node_body,
        grid=(VP // BM,),
        in_specs=[
            pl.BlockSpec((2, BM, 64), lambda i: (0, i, 0)),
            pl.BlockSpec((BM, 128), lambda i: (i, 0)),
            pl.BlockSpec((BM, 128), lambda i: (i, 0)),
            pl.BlockSpec((BM, 64), lambda i: (i, 0)),
            pl.BlockSpec((BM, 64), lambda i: (i, 0)),
            pl.BlockSpec((BM, 64), lambda i: (i, 0)),
            pl.BlockSpec((BM, 2), lambda i: (i, 0)),
            pl.BlockSpec((64, 128), lambda i: (0, 0)),
            pl.BlockSpec((128, 384), lambda i: (0, 0)),
            pl.BlockSpec((128, 384), lambda i: (0, 0)),
            pl.BlockSpec((1, 384), lambda i: (0, 0)),
            pl.BlockSpec((1, 384), lambda i: (0, 0)),
            pl.BlockSpec((128, 64), lambda i: (0, 0)),
            pl.BlockSpec((64, 64), lambda i: (0, 0)),
            pl.BlockSpec((1, 64), lambda i: (0, 0)),
            pl.BlockSpec((64, 128), lambda i: (0, 0)),
            pl.BlockSpec((128, 64), lambda i: (0, 0)),
            pl.BlockSpec((128, 64), lambda i: (0, 0)),
        ],
        out_specs=[
            pl.BlockSpec((BM, 128), lambda i: (i, 0)),
            pl.BlockSpec((BM, 64), lambda i: (i, 0)),
            pl.BlockSpec((BM, 64), lambda i: (i, 0)),
            pl.BlockSpec((BM, 1), lambda i: (i, 0)),
            pl.BlockSpec((1, 1), lambda i: (0, 0)),
        ],
        out_shape=[
            jax.ShapeDtypeStruct((VP, 128), f32),
            jax.ShapeDtypeStruct((VP, 64), bf16),
            jax.ShapeDtypeStruct((VP, 64), bf16),
            jax.ShapeDtypeStruct((VP, 1), f32),
            jax.ShapeDtypeStruct((1, 1), f32),
        ],
    )(s2, st, dvec, abias, bbias, obias, tpad, W3T, WihT, WhhT,
      bihr, bhhr, O1sT, O2T, ob2r, O3Tp, W1aT, W1bT)


def kernel(J_msg, b, msg_node, idx_msg_edge, target, W1, b1, W2, b2, W3, b3,
           Wih, Whh, bih, bhh, O1, ob1, O2, ob2, O3, ob3):
    del idx_msg_edge
    perm = jnp.array([32 * g + x for g in range(2)
                      for k in range(16) for x in (k, k + 16)], dtype=jnp.int32)
    W1aT = W1[:, 0:128].T[:, perm]
    W1bT = W1[:, 132:260].T[:, perm]
    u = W1[:, 128] - W1[:, 129]
    v = W1[:, 261] - W1[:, 260]
    w = (W1[:, 130] - W1[:, 131]) + (W1[:, 263] - W1[:, 262])
    bp = jnp.pad(b, ((0, VP - V), (0, 0)))
    abias = (bp * u[None, :])[:, perm]
    bbias = (bp * v[None, :] + b1[None, :])[:, perm]
    obias = bp * (O1[:, 128] - O1[:, 129])[None, :] + ob1[None, :]
    tpad = jnp.pad(target, ((0, VP - V), (0, 0)), constant_values=1.0)
    W2T = W2.T
    W2bd = jnp.zeros((128, 128), f32).at[:64, :64].set(W2T).at[64:, 64:].set(W2T)
    b2d = jnp.concatenate([b2, b2])[None, :]
    z64 = jnp.zeros((64,), f32)
    W3T = W3.T
    WihT = Wih.T
    WhhT = Whh.T
    O1sT = O1[:, 0:128].T
    O2T = O2.T
    O3Tp = jnp.pad(O3.T, ((0, 0), (0, 128 - 2)))
    bihr = bih[None, :]
    bhhr = bhh[None, :]
    ob2r = ob2[None, :]
    wa = jnp.concatenate([w, z64])[None, :]
    wb = jnp.concatenate([z64, w])[None, :]
    ii = msg_node[:, 0].astype(jnp.int32)
    io = msg_node[:, 1].astype(jnp.int32)
    j2 = J_msg.reshape(EH, 2)
    ioe = io[0::2]
    ioo = io[1::2]

    state = jnp.zeros((VP, H), f32)
    ap = abias.astype(bf16)
    bpp = bbias.astype(bf16)
    dvec = None
    ys = []
    lsum = None
    for t in range(NPROP):
        m1raw = _sc_gather(ap, bpp, ii, io)
        m2 = _tc_edge(m1raw, j2, W2bd, b2d, wa, wb)
        if t == 0:
            deg2 = _sc_deg(io)
            deg = deg2[0, :, 0] + deg2[1, :, 0]
            dvec = deg[:, None] * b3[None, :]
        s2 = _sc_scatter(m2, ioe, ioo)
        state, ap, bpp, y, lsum = _tc_node(
            s2, state, dvec, abias, bbias, obias, tpad, W3T, WihT, WhhT,
            bihr, bhhr, O1sT, O2T, ob2r, O3Tp, W1aT, W1bT)
        ys.append(y)
    y_step = jnp.concatenate(ys, axis=1)[:V, :]
    loss = (lsum[0, 0] / jnp.float32(V)).astype(f32)
    return (y_step, loss)
